# baseline (speedup 1.0000x reference)
import jax
import jax.numpy as jnp
from jax import lax
from jax.experimental import pallas as pl
from jax.experimental.pallas import tpu as pltpu
from jax.experimental.pallas import tpu_sc as plsc

_D = 1024
_W = _D // 2
_NC = 2
_NS = 16
_NW = _NC * _NS
_CH = 64


def _sc_gather(idx_flat, table3, n):
    b_per_w = n // _NW
    n_chunk = b_per_w // _CH
    mesh = plsc.VectorSubcoreMesh(core_axis_name="core", subcore_axis_name="subcore")

    @pl.kernel(
        out_type=jax.ShapeDtypeStruct((n, _W), table3.dtype),
        mesh=mesh,
        scratch_types=[
            pltpu.VMEM((b_per_w,), jnp.int32),
            pltpu.VMEM((_CH, _W), jnp.int32),
            pltpu.VMEM((_CH, _W), jnp.int32),
            pltpu.VMEM((_CH, _W), jnp.int32),
            pltpu.SemaphoreType.DMA,
            pltpu.SemaphoreType.DMA,
            pltpu.SemaphoreType.DMA,
            pltpu.SemaphoreType.DMA,
            pltpu.SemaphoreType.DMA,
            pltpu.SemaphoreType.DMA,
        ],
    )
    def gather_kernel(
        table_hbm, idx_hbm, out_hbm, idx_v,
        rows0, rows1, rows2, gsem0, gsem1, gsem2, ssem0, ssem1, ssem2,
    ):
        wid = lax.axis_index("subcore") * _NC + lax.axis_index("core")
        base = wid * b_per_w
        pltpu.sync_copy(idx_hbm.at[pl.ds(base, b_per_w)], idx_v)

        rows = [rows0, rows1, rows2]
        gsem = [gsem0, gsem1, gsem2]
        ssem = [ssem0, ssem1, ssem2]

        def wait_gather(b):
            pltpu.make_async_copy(table_hbm.at[pl.ds(0, _CH)], rows[b], gsem[b]).wait()

        def wait_store(b):
            pltpu.make_async_copy(rows[b], out_hbm.at[pl.ds(base, _CH)], ssem[b]).wait()

        @pl.loop(0, n_chunk + 3, step=3)
        def _(c):
            for b in range(3):
                k = c + b
                bj = (b + 1) % 3

                @pl.when(jnp.logical_and(k >= 3, k < n_chunk))
                def _():
                    wait_store(b)

                @pl.when(k < n_chunk)
                def _():
                    pltpu.async_copy(
                        table_hbm.at[idx_v.at[pl.ds(k * _CH, _CH)]], rows[b], gsem[b]
                    )

                j = k - 2

                @pl.when(jnp.logical_and(j >= 0, j < n_chunk))
                def _():
                    wait_gather(bj)
                    pltpu.async_copy(
                        rows[bj], out_hbm.at[pl.ds(base + j * _CH, _CH)], ssem[bj]
                    )

        for b in range(3):
            wait_store(b)

    return gather_kernel(table3, idx_flat)


def kernel(position_ids, pos_emb):
    b, s = position_ids.shape
    n = b * s
    idx_flat = position_ids.reshape(n).astype(jnp.int32)
    table_i32 = jax.lax.bitcast_convert_type(
        pos_emb.astype(jnp.bfloat16).reshape(-1, _W, 2), jnp.int32
    )
    out = _sc_gather(idx_flat, table_i32, n)
    out_bf16 = jax.lax.bitcast_convert_type(out, jnp.bfloat16)
    return out_bf16.reshape(b, s, _D).astype(jnp.float32)


# speedup vs baseline: 4.4841x; 4.4841x over previous
import jax
import jax.numpy as jnp
from jax import lax
from jax.experimental import pallas as pl
from jax.experimental.pallas import tpu as pltpu
from jax.experimental.pallas import tpu_sc as plsc

_D = 1024
_W = _D // 2
_NC = 2
_NS = 16
_NW = _NC * _NS
_CH = 64


def _sc_gather(idx_flat, table3, n):
    b_per_w = n // _NW
    n_chunk = b_per_w // _CH
    mesh = plsc.VectorSubcoreMesh(core_axis_name="core", subcore_axis_name="subcore")

    @pl.kernel(
        out_type=jax.ShapeDtypeStruct((n, _W), table3.dtype),
        mesh=mesh,
        scratch_types=[
            pltpu.VMEM((b_per_w,), jnp.int32),
            pltpu.VMEM((_CH, _W), jnp.int32),
            pltpu.VMEM((_CH, _W), jnp.int32),
            pltpu.VMEM((_CH, _W), jnp.int32),
            pltpu.SemaphoreType.DMA,
            pltpu.SemaphoreType.DMA,
            pltpu.SemaphoreType.DMA,
            pltpu.SemaphoreType.DMA,
            pltpu.SemaphoreType.DMA,
            pltpu.SemaphoreType.DMA,
        ],
    )
    def gather_kernel(
        table_hbm, idx_hbm, out_hbm, idx_v,
        rows0, rows1, rows2, gsem0, gsem1, gsem2, ssem0, ssem1, ssem2,
    ):
        wid = lax.axis_index("subcore") * _NC + lax.axis_index("core")
        base = wid * b_per_w
        pltpu.sync_copy(idx_hbm.at[pl.ds(base, b_per_w)], idx_v)

        rows = [rows0, rows1, rows2]
        gsem = [gsem0, gsem1, gsem2]
        ssem = [ssem0, ssem1, ssem2]

        def wait_gather(b):
            pltpu.make_async_copy(table_hbm.at[pl.ds(0, _CH)], rows[b], gsem[b]).wait()

        def wait_store(b):
            pltpu.make_async_copy(rows[b], out_hbm.at[pl.ds(base, _CH)], ssem[b]).wait()

        @pl.loop(0, n_chunk + 3, step=3)
        def _(c):
            for b in range(3):
                k = c + b
                bj = (b + 1) % 3

                @pl.when(jnp.logical_and(k >= 3, k < n_chunk))
                def _():
                    wait_store(b)

                @pl.when(k < n_chunk)
                def _():
                    pltpu.async_copy(
                        table_hbm.at[idx_v.at[pl.ds(k * _CH, _CH)]], rows[b], gsem[b]
                    )

                j = k - 2

                @pl.when(jnp.logical_and(j >= 0, j < n_chunk))
                def _():
                    wait_gather(bj)
                    pltpu.async_copy(
                        rows[bj], out_hbm.at[pl.ds(base + j * _CH, _CH)], ssem[bj]
                    )

        for b in range(3):
            wait_store(b)

    return gather_kernel(table3, idx_flat)


def _rne_bf16_bits(f32_block):
    u = jax.lax.bitcast_convert_type(f32_block, jnp.uint32)
    return (u + jnp.uint32(0x7FFF) + ((u >> 16) & jnp.uint32(1))) >> 16


def _pack_kernel_body(x_ref, o_ref):
    lo = _rne_bf16_bits(x_ref[:, :_W])
    hi = _rne_bf16_bits(x_ref[:, _W:])
    o_ref[...] = (lo | (hi << 16)).astype(jnp.int32)


def _unpack_kernel_body(g_ref, o_ref):
    g = g_ref[...].astype(jnp.uint32)
    o_ref[:, :_W] = jax.lax.bitcast_convert_type(g << 16, jnp.float32)
    o_ref[:, _W:] = jax.lax.bitcast_convert_type(
        g & jnp.uint32(0xFFFF0000), jnp.float32
    )


def _pack_table(pos_emb):
    v = pos_emb.shape[0]
    blk = 512
    return pl.pallas_call(
        _pack_kernel_body,
        out_shape=jax.ShapeDtypeStruct((v, _W), jnp.int32),
        grid=(v // blk,),
        in_specs=[pl.BlockSpec((blk, _D), lambda i: (i, 0))],
        out_specs=pl.BlockSpec((blk, _W), lambda i: (i, 0)),
        compiler_params=pltpu.CompilerParams(
            dimension_semantics=("parallel",)
        ),
    )(pos_emb)


def _unpack_rows(g):
    n = g.shape[0]
    blk = 512
    return pl.pallas_call(
        _unpack_kernel_body,
        out_shape=jax.ShapeDtypeStruct((n, _D), jnp.float32),
        grid=(n // blk,),
        in_specs=[pl.BlockSpec((blk, _W), lambda i: (i, 0))],
        out_specs=pl.BlockSpec((blk, _D), lambda i: (i, 0)),
        compiler_params=pltpu.CompilerParams(
            dimension_semantics=("parallel",)
        ),
    )(g)


def kernel(position_ids, pos_emb):
    b, s = position_ids.shape
    n = b * s
    idx_flat = position_ids.reshape(n).astype(jnp.int32)
    table_i32 = _pack_table(pos_emb)
    out = _sc_gather(idx_flat, table_i32, n)
    return _unpack_rows(out).reshape(b, s, _D)


# speedup vs baseline: 6.7295x; 1.5008x over previous
import jax
import jax.numpy as jnp
from jax import lax
from jax.experimental import pallas as pl
from jax.experimental.pallas import tpu as pltpu
from jax.experimental.pallas import tpu_sc as plsc

_D = 1024
_NC = 2
_NS = 16
_NW = _NC * _NS
_CH = 32


def _sc_gather(idx_flat, pos_emb, n):
    b_per_w = n // _NW
    n_chunk = b_per_w // _CH
    mesh = plsc.VectorSubcoreMesh(core_axis_name="core", subcore_axis_name="subcore")

    @pl.kernel(
        out_type=jax.ShapeDtypeStruct((n, _D), pos_emb.dtype),
        mesh=mesh,
        scratch_types=[
            pltpu.VMEM((b_per_w,), jnp.int32),
            pltpu.VMEM((_CH, _D), jnp.float32),
            pltpu.VMEM((_CH, _D), jnp.float32),
            pltpu.VMEM((_CH, _D), jnp.float32),
            pltpu.SemaphoreType.DMA,
            pltpu.SemaphoreType.DMA,
            pltpu.SemaphoreType.DMA,
            pltpu.SemaphoreType.DMA,
            pltpu.SemaphoreType.DMA,
            pltpu.SemaphoreType.DMA,
        ],
    )
    def gather_kernel(
        table_hbm, idx_hbm, out_hbm, idx_v,
        rows0, rows1, rows2, gsem0, gsem1, gsem2, ssem0, ssem1, ssem2,
    ):
        wid = lax.axis_index("subcore") * _NC + lax.axis_index("core")
        base = wid * b_per_w
        pltpu.sync_copy(idx_hbm.at[pl.ds(base, b_per_w)], idx_v)

        rows = [rows0, rows1, rows2]
        gsem = [gsem0, gsem1, gsem2]
        ssem = [ssem0, ssem1, ssem2]

        def wait_gather(b):
            pltpu.make_async_copy(table_hbm.at[pl.ds(0, _CH)], rows[b], gsem[b]).wait()

        def wait_store(b):
            pltpu.make_async_copy(rows[b], out_hbm.at[pl.ds(base, _CH)], ssem[b]).wait()

        @pl.loop(0, n_chunk + 3, step=3)
        def _(c):
            for b in range(3):
                k = c + b
                bj = (b + 1) % 3

                @pl.when(jnp.logical_and(k >= 3, k < n_chunk))
                def _():
                    wait_store(b)

                @pl.when(k < n_chunk)
                def _():
                    pltpu.async_copy(
                        table_hbm.at[idx_v.at[pl.ds(k * _CH, _CH)]], rows[b], gsem[b]
                    )

                j = k - 2

                @pl.when(jnp.logical_and(j >= 0, j < n_chunk))
                def _():
                    wait_gather(bj)
                    pltpu.async_copy(
                        rows[bj], out_hbm.at[pl.ds(base + j * _CH, _CH)], ssem[bj]
                    )

        for b in range(3):
            wait_store(b)

    return gather_kernel(pos_emb, idx_flat)


def kernel(position_ids, pos_emb):
    b, s = position_ids.shape
    n = b * s
    idx_flat = position_ids.reshape(n).astype(jnp.int32)
    out = _sc_gather(idx_flat, pos_emb, n)
    return out.reshape(b, s, _D)
